# baseline (speedup 1.0000x reference)
import functools

import jax
import jax.numpy as jnp
from jax import lax
from jax.experimental import pallas as pl
from jax.experimental.pallas import tpu as pltpu
from jax.experimental.pallas import tpu_sc as plsc

N = 10000
E = 320000
D = 128
H = 128
L = 64
MAXLS = 10.0

NC, NS = 2, 16
NW = NC * NS
EPT = E // NW
CH = 80
NCHUNK = EPT // CH
NPT = N // NS

_MESH = plsc.VectorSubcoreMesh(core_axis_name="c", subcore_axis_name="s",
                               num_cores=NC, num_subcores=NS)
_SC_PARAMS = pltpu.CompilerParams(needs_layout_passes=False,
                                  use_tc_tiling_on_sc=False)


def _wid():
    return lax.axis_index("c") * NS + lax.axis_index("s")


def _bounce_idx(src_flat, i, dst):
    for k in range(0, CH, 16):
        dst[pl.ds(k, 16)] = src_flat[pl.ds(i * CH + k, 16)]


@functools.partial(
    pl.kernel,
    out_type=jax.ShapeDtypeStruct((NC, N * 16), jnp.float32),
    mesh=_MESH,
    scratch_types=[
        pltpu.VMEM((EPT,), jnp.int32),
        pltpu.VMEM((CH,), jnp.int32),
        pltpu.VMEM((CH,), jnp.int32),
        pltpu.VMEM((CH, 16), jnp.float32),
        pltpu.VMEM((NPT, 16), jnp.float32),
        pltpu.VMEM((NPT * 16,), jnp.float32),
        pltpu.VMEM_SHARED((N, 16), jnp.float32),
        pltpu.SemaphoreType.DMA,
        pltpu.SemaphoreType.DMA,
    ],
    compiler_params=_SC_PARAMS,
)
def _deg_kernel(col_hbm, out_hbm,
                cflat, cidx0, cidx1, ones_v, degv, obuf, acc_sh, sem0, sem1):
    c = lax.axis_index("c")
    s = lax.axis_index("s")
    ones16 = jnp.ones((16,), jnp.float32)

    @pl.loop(0, CH)
    def _(r):
        ones_v[r, pl.ds(0, 16)] = ones16

    zeros16 = jnp.zeros((16,), jnp.float32)

    @pl.loop(0, NPT)
    def _(r):
        degv[r, pl.ds(0, 16)] = zeros16

    pltpu.sync_copy(degv, acc_sh.at[pl.ds(s * NPT, NPT)])
    pltpu.sync_copy(col_hbm.at[pl.ds(_wid() * EPT, EPT)], cflat)
    plsc.subcore_barrier()

    @pl.loop(0, NCHUNK - 1, step=2)
    def _(i):
        _bounce_idx(cflat, i, cidx0)
        d0 = pltpu.async_copy(ones_v, acc_sh.at[cidx0], sem0, add=True)
        _bounce_idx(cflat, i + 1, cidx1)
        d1 = pltpu.async_copy(ones_v, acc_sh.at[cidx1], sem1, add=True)
        d0.wait()
        d1.wait()

    _bounce_idx(cflat, NCHUNK - 1, cidx0)
    pltpu.sync_copy(ones_v, acc_sh.at[cidx0], add=True)

    plsc.subcore_barrier()

    pltpu.sync_copy(acc_sh.at[pl.ds(s * NPT, NPT)], degv)

    @pl.loop(0, NPT)
    def _(r):
        obuf[pl.ds(r * 16, 16)] = degv[r, pl.ds(0, 16)]

    pltpu.sync_copy(obuf, out_hbm.at[c].at[pl.ds(s * NPT * 16, NPT * 16)])


def _make_prop(F):
    @functools.partial(
        pl.kernel,
        out_type=jax.ShapeDtypeStruct((NC, N, F), jnp.float32),
        mesh=_MESH,
        scratch_types=[
            pltpu.VMEM((EPT,), jnp.int32),
            pltpu.VMEM((EPT,), jnp.int32),
            [pltpu.VMEM((CH,), jnp.int32)] * 2,
            [pltpu.VMEM((CH,), jnp.int32)] * 2,
            [pltpu.VMEM((CH, F), jnp.float32)] * 2,
            pltpu.VMEM_SHARED((N, F), jnp.float32),
            [pltpu.SemaphoreType.DMA] * 2,
            [pltpu.SemaphoreType.DMA] * 2,
        ],
        compiler_params=_SC_PARAMS,
    )
    def _prop_kernel(g_hbm, row_hbm, col_hbm, zeros_hbm, out_hbm,
                     rflat, cflat, ridxd, cidxd, gbuf,
                     acc_sh, semg, sems):
        c = lax.axis_index("c")
        s = lax.axis_index("s")
        base = _wid() * EPT

        @pl.when(s == 0)
        def _():
            pltpu.sync_copy(zeros_hbm, acc_sh)

        pltpu.sync_copy(row_hbm.at[pl.ds(base, EPT)], rflat)
        pltpu.sync_copy(col_hbm.at[pl.ds(base, EPT)], cflat)
        plsc.subcore_barrier()

        def fire_gather(i, k):
            _bounce_idx(rflat, i, ridxd[k])
            pltpu.async_copy(g_hbm.at[ridxd[k]], gbuf[k], semg[k])

        def wait_gather(k):
            pltpu.make_async_copy(g_hbm.at[ridxd[k]], gbuf[k],
                                  semg[k]).wait()

        def fire_scatter(i, k):
            _bounce_idx(cflat, i, cidxd[k])
            pltpu.async_copy(gbuf[k], acc_sh.at[cidxd[k]], sems[k], add=True)

        def wait_scatter(k):
            pltpu.make_async_copy(gbuf[k], acc_sh.at[cidxd[k]],
                                  sems[k]).wait()

        fire_gather(0, 0)
        fire_gather(1, 1)

        @pl.loop(0, NCHUNK - 1, step=2)
        def _(i):
            wait_gather(0)
            fire_scatter(i, 0)
            wait_gather(1)
            fire_scatter(i + 1, 1)
            wait_scatter(0)
            fire_gather(i + 2, 0)

            @pl.when(i + 3 < NCHUNK)
            def _():
                wait_scatter(1)
                fire_gather(i + 3, 1)

        wait_gather(0)
        fire_scatter(NCHUNK - 1, 0)
        wait_scatter(1)
        wait_scatter(0)

        plsc.subcore_barrier()

        @pl.when(s == 0)
        def _():
            pltpu.sync_copy(acc_sh, out_hbm.at[c])

    return _prop_kernel


_prop128 = _make_prop(H)


@functools.partial(
    pl.kernel,
    out_type=jax.ShapeDtypeStruct((E,), jnp.float32),
    mesh=_MESH,
    scratch_types=[
        pltpu.VMEM((EPT,), jnp.int32),
        pltpu.VMEM((EPT,), jnp.int32),
        [pltpu.VMEM((CH,), jnp.int32)] * 3,
        [pltpu.VMEM((CH,), jnp.int32)] * 3,
        [pltpu.VMEM((CH, 128), jnp.float32)] * 3,
        [pltpu.VMEM((CH, 128), jnp.float32)] * 3,
        pltpu.VMEM((EPT,), jnp.float32),
        [pltpu.SemaphoreType.DMA] * 3,
        [pltpu.SemaphoreType.DMA] * 3,
    ],
    compiler_params=_SC_PARAMS,
)
def _decode_kernel(z_hbm, row_hbm, col_hbm, out_hbm,
                   rflat, cflat, ridxd, cidxd, zr, zc, obuf, semr, semc):
    base = _wid() * EPT
    pltpu.sync_copy(row_hbm.at[pl.ds(base, EPT)], rflat)
    pltpu.sync_copy(col_hbm.at[pl.ds(base, EPT)], cflat)

    def fire(i, k):
        _bounce_idx(rflat, i, ridxd[k])
        pltpu.async_copy(z_hbm.at[ridxd[k]], zr[k], semr[k])
        _bounce_idx(cflat, i, cidxd[k])
        pltpu.async_copy(z_hbm.at[cidxd[k]], zc[k], semc[k])

    def wait(k):
        pltpu.make_async_copy(z_hbm.at[ridxd[k]], zr[k], semr[k]).wait()
        pltpu.make_async_copy(z_hbm.at[cidxd[k]], zc[k], semc[k]).wait()

    def compute(i, zr, zc):
        for e0 in range(0, CH, 16):
            vec = jnp.zeros((16,), jnp.float32)
            for j in range(16):
                e = e0 + j
                p = zr[e, pl.ds(0, 16)] * zc[e, pl.ds(0, 16)]
                for q in range(16, L, 16):
                    p = p + zr[e, pl.ds(q, 16)] * zc[e, pl.ds(q, 16)]
                sval = jnp.sum(p)
                vec = jnp.where(lax.iota(jnp.int32, 16) == j, sval, vec)
            sig = 1.0 / (1.0 + jnp.exp(-vec))
            obuf[pl.ds(i * CH + e0, 16)] = sig

    for k in range(3):
        fire(k, k)

    @pl.loop(0, NCHUNK - 2, step=3)
    def _(i):
        for k in range(3):
            wait(k)
            compute(i + k, zr[k], zc[k])

            @pl.when(i + k + 3 < NCHUNK)
            def _(k=k):
                fire(i + k + 3, k)

    wait(0)
    compute(NCHUNK - 2, zr[0], zc[0])
    wait(1)
    compute(NCHUNK - 1, zr[1], zc[1])

    pltpu.sync_copy(obuf, out_hbm.at[pl.ds(base, EPT)])


def _tc1_body(deg_ref, x_ref, w1_ref, t_ref, dis_ref):
    deg = deg_ref[0] + deg_ref[1]
    dis = jnp.where(deg > 0, lax.rsqrt(jnp.maximum(deg, 1.0)), 0.0)
    dis_ref[...] = dis
    xw = jnp.dot(x_ref[...], w1_ref[...], preferred_element_type=jnp.float32)
    t_ref[...] = xw * dis[:, None]


def _tc2_body(s1p_ref, dis_ref, g_ref):
    s1 = s1p_ref[0] + s1p_ref[1]
    dis = dis_ref[...]
    h1 = jnp.maximum(dis[:, None] * s1, 0.0)
    g_ref[...] = dis[:, None] * h1


def _tc3_body(s2p_ref, dis_ref, eps_ref, wmu_ref, wls_ref, z_ref):
    s2 = s2p_ref[0] + s2p_ref[1]
    dis = dis_ref[...]
    mu = dis[:, None] * jnp.dot(s2, wmu_ref[...],
                                preferred_element_type=jnp.float32)
    lg = jnp.minimum(dis[:, None] * jnp.dot(s2, wls_ref[...],
                                            preferred_element_type=jnp.float32),
                     MAXLS)
    z_ref[:, 0:L] = mu + eps_ref[...] * jnp.exp(lg)
    z_ref[:, L:128] = jnp.zeros((N, 128 - L), jnp.float32)


def kernel(x, edge_index, eps, W1, W_mu, W_logstd):
    row = edge_index[0]
    col = edge_index[1]

    degf = _deg_kernel(col)
    deg2 = degf.reshape(NC, N, 16)[:, :, 0]

    t, dis = pl.pallas_call(
        _tc1_body,
        out_shape=(jax.ShapeDtypeStruct((N, H), jnp.float32),
                   jax.ShapeDtypeStruct((N,), jnp.float32)),
    )(deg2, x, W1)

    zerosF = jnp.zeros((N, H), jnp.float32)
    s1p = _prop128(t, row, col, zerosF)

    g = pl.pallas_call(
        _tc2_body,
        out_shape=jax.ShapeDtypeStruct((N, H), jnp.float32),
    )(s1p, dis)

    s2p = _prop128(g, row, col, zerosF)

    z = pl.pallas_call(
        _tc3_body,
        out_shape=jax.ShapeDtypeStruct((N, 128), jnp.float32),
    )(s2p, dis, eps, W_mu, W_logstd)

    return _decode_kernel(z, row, col)


# speedup vs baseline: 1.1261x; 1.1261x over previous
import functools

import jax
import jax.numpy as jnp
from jax import lax
from jax.experimental import pallas as pl
from jax.experimental.pallas import tpu as pltpu
from jax.experimental.pallas import tpu_sc as plsc

N = 10000
E = 320000
D = 128
H = 128
L = 64
MAXLS = 10.0

NC, NS = 2, 16
NW = NC * NS
EPT = E // NW
CH = 80
NCHUNK = EPT // CH
NPT = N // NS

_MESH = plsc.VectorSubcoreMesh(core_axis_name="c", subcore_axis_name="s",
                               num_cores=NC, num_subcores=NS)
_SC_PARAMS = pltpu.CompilerParams(needs_layout_passes=False,
                                  use_tc_tiling_on_sc=False)


def _wid():
    return lax.axis_index("c") * NS + lax.axis_index("s")


def _bounce_idx(src_flat, i, dst):
    for k in range(0, CH, 16):
        dst[pl.ds(k, 16)] = src_flat[pl.ds(i * CH + k, 16)]


@functools.partial(
    pl.kernel,
    out_type=jax.ShapeDtypeStruct((NC, N * 16), jnp.float32),
    mesh=_MESH,
    scratch_types=[
        pltpu.VMEM((EPT,), jnp.int32),
        pltpu.VMEM((CH,), jnp.int32),
        pltpu.VMEM((CH,), jnp.int32),
        pltpu.VMEM((CH, 16), jnp.float32),
        pltpu.VMEM((NPT, 16), jnp.float32),
        pltpu.VMEM((NPT * 16,), jnp.float32),
        pltpu.VMEM_SHARED((N, 16), jnp.float32),
        pltpu.SemaphoreType.DMA,
        pltpu.SemaphoreType.DMA,
    ],
    compiler_params=_SC_PARAMS,
)
def _deg_kernel(col_hbm, out_hbm,
                cflat, cidx0, cidx1, ones_v, degv, obuf, acc_sh, sem0, sem1):
    c = lax.axis_index("c")
    s = lax.axis_index("s")
    ones16 = jnp.ones((16,), jnp.float32)

    @pl.loop(0, CH)
    def _(r):
        ones_v[r, pl.ds(0, 16)] = ones16

    zeros16 = jnp.zeros((16,), jnp.float32)

    @pl.loop(0, NPT)
    def _(r):
        degv[r, pl.ds(0, 16)] = zeros16

    pltpu.sync_copy(degv, acc_sh.at[pl.ds(s * NPT, NPT)])
    pltpu.sync_copy(col_hbm.at[pl.ds(_wid() * EPT, EPT)], cflat)
    plsc.subcore_barrier()

    @pl.loop(0, NCHUNK - 1, step=2)
    def _(i):
        _bounce_idx(cflat, i, cidx0)
        d0 = pltpu.async_copy(ones_v, acc_sh.at[cidx0], sem0, add=True)
        _bounce_idx(cflat, i + 1, cidx1)
        d1 = pltpu.async_copy(ones_v, acc_sh.at[cidx1], sem1, add=True)
        d0.wait()
        d1.wait()

    _bounce_idx(cflat, NCHUNK - 1, cidx0)
    pltpu.sync_copy(ones_v, acc_sh.at[cidx0], add=True)

    plsc.subcore_barrier()

    pltpu.sync_copy(acc_sh.at[pl.ds(s * NPT, NPT)], degv)

    @pl.loop(0, NPT)
    def _(r):
        obuf[pl.ds(r * 16, 16)] = degv[r, pl.ds(0, 16)]

    pltpu.sync_copy(obuf, out_hbm.at[c].at[pl.ds(s * NPT * 16, NPT * 16)])


def _make_prop(F):
    @functools.partial(
        pl.kernel,
        out_type=jax.ShapeDtypeStruct((NC, N, F), jnp.float32),
        mesh=_MESH,
        scratch_types=[
            pltpu.VMEM((EPT,), jnp.int32),
            pltpu.VMEM((EPT,), jnp.int32),
            [pltpu.VMEM((CH,), jnp.int32)] * 2,
            [pltpu.VMEM((CH,), jnp.int32)] * 2,
            [pltpu.VMEM((CH, F), jnp.float32)] * 2,
            pltpu.VMEM_SHARED((N, F), jnp.float32),
            [pltpu.SemaphoreType.DMA] * 2,
            [pltpu.SemaphoreType.DMA] * 2,
        ],
        compiler_params=_SC_PARAMS,
    )
    def _prop_kernel(g_hbm, row_hbm, col_hbm, zeros_hbm, out_hbm,
                     rflat, cflat, ridxd, cidxd, gbuf,
                     acc_sh, semg, sems):
        c = lax.axis_index("c")
        s = lax.axis_index("s")
        base = _wid() * EPT

        @pl.when(s == 0)
        def _():
            pltpu.sync_copy(zeros_hbm, acc_sh)

        pltpu.sync_copy(row_hbm.at[pl.ds(base, EPT)], rflat)
        pltpu.sync_copy(col_hbm.at[pl.ds(base, EPT)], cflat)
        plsc.subcore_barrier()

        def fire_gather(i, k):
            _bounce_idx(rflat, i, ridxd[k])
            pltpu.async_copy(g_hbm.at[ridxd[k]], gbuf[k], semg[k])

        def wait_gather(k):
            pltpu.make_async_copy(g_hbm.at[ridxd[k]], gbuf[k],
                                  semg[k]).wait()

        def fire_scatter(i, k):
            _bounce_idx(cflat, i, cidxd[k])
            pltpu.async_copy(gbuf[k], acc_sh.at[cidxd[k]], sems[k], add=True)

        def wait_scatter(k):
            pltpu.make_async_copy(gbuf[k], acc_sh.at[cidxd[k]],
                                  sems[k]).wait()

        def sync_scatter(i, k):
            _bounce_idx(cflat, i, cidxd[k])
            pltpu.sync_copy(gbuf[k], acc_sh.at[cidxd[k]], add=True)

        fire_gather(0, 0)
        fire_gather(1, 1)

        @pl.loop(0, NCHUNK - 1, step=2)
        def _(i):
            wait_gather(0)
            sync_scatter(i, 0)
            fire_gather(i + 2, 0)
            wait_gather(1)
            sync_scatter(i + 1, 1)

            @pl.when(i + 3 < NCHUNK)
            def _():
                fire_gather(i + 3, 1)

        wait_gather(0)
        sync_scatter(NCHUNK - 1, 0)

        plsc.subcore_barrier()

        @pl.when(s == 0)
        def _():
            pltpu.sync_copy(acc_sh, out_hbm.at[c])

    return _prop_kernel


_prop128 = _make_prop(H)


@functools.partial(
    pl.kernel,
    out_type=jax.ShapeDtypeStruct((E,), jnp.float32),
    mesh=_MESH,
    scratch_types=[
        pltpu.VMEM((EPT,), jnp.int32),
        pltpu.VMEM((EPT,), jnp.int32),
        [pltpu.VMEM((CH,), jnp.int32)] * 3,
        [pltpu.VMEM((CH,), jnp.int32)] * 3,
        [pltpu.VMEM((CH, L), jnp.float32)] * 3,
        [pltpu.VMEM((CH, L), jnp.float32)] * 3,
        pltpu.VMEM((EPT,), jnp.float32),
        [pltpu.SemaphoreType.DMA] * 3,
        [pltpu.SemaphoreType.DMA] * 3,
    ],
    compiler_params=_SC_PARAMS,
)
def _decode_kernel(z_hbm, row_hbm, col_hbm, out_hbm,
                   rflat, cflat, ridxd, cidxd, zr, zc, obuf, semr, semc):
    base = _wid() * EPT
    pltpu.sync_copy(row_hbm.at[pl.ds(base, EPT)], rflat)
    pltpu.sync_copy(col_hbm.at[pl.ds(base, EPT)], cflat)

    def fire(i, k):
        _bounce_idx(rflat, i, ridxd[k])
        pltpu.async_copy(z_hbm.at[ridxd[k]], zr[k], semr[k])
        _bounce_idx(cflat, i, cidxd[k])
        pltpu.async_copy(z_hbm.at[cidxd[k]], zc[k], semc[k])

    def wait(k):
        pltpu.make_async_copy(z_hbm.at[ridxd[k]], zr[k], semr[k]).wait()
        pltpu.make_async_copy(z_hbm.at[cidxd[k]], zc[k], semc[k]).wait()

    def compute(i, zr, zc):
        for e0 in range(0, CH, 16):
            vec = jnp.zeros((16,), jnp.float32)
            for j in range(16):
                e = e0 + j
                p = zr[e, pl.ds(0, 16)] * zc[e, pl.ds(0, 16)]
                for q in range(16, L, 16):
                    p = p + zr[e, pl.ds(q, 16)] * zc[e, pl.ds(q, 16)]
                sval = jnp.sum(p)
                vec = jnp.where(lax.iota(jnp.int32, 16) == j, sval, vec)
            sig = 1.0 / (1.0 + jnp.exp(-vec))
            obuf[pl.ds(i * CH + e0, 16)] = sig

    for k in range(3):
        fire(k, k)

    @pl.loop(0, NCHUNK - 2, step=3)
    def _(i):
        for k in range(3):
            wait(k)
            compute(i + k, zr[k], zc[k])

            @pl.when(i + k + 3 < NCHUNK)
            def _(k=k):
                fire(i + k + 3, k)

    wait(0)
    compute(NCHUNK - 2, zr[0], zc[0])
    wait(1)
    compute(NCHUNK - 1, zr[1], zc[1])

    pltpu.sync_copy(obuf, out_hbm.at[pl.ds(base, EPT)])


def _tc1_body(deg_ref, x_ref, w1_ref, t_ref, dis_ref):
    deg = deg_ref[0] + deg_ref[1]
    dis = jnp.where(deg > 0, lax.rsqrt(jnp.maximum(deg, 1.0)), 0.0)
    dis_ref[...] = dis
    xw = jnp.dot(x_ref[...], w1_ref[...], preferred_element_type=jnp.float32)
    t_ref[...] = xw * dis[:, None]


def _tc2_body(s1p_ref, dis_ref, g_ref):
    s1 = s1p_ref[0] + s1p_ref[1]
    dis = dis_ref[...]
    h1 = jnp.maximum(dis[:, None] * s1, 0.0)
    g_ref[...] = dis[:, None] * h1


def _tc3_body(s2p_ref, dis_ref, eps_ref, wmu_ref, wls_ref, z_ref):
    s2 = s2p_ref[0] + s2p_ref[1]
    dis = dis_ref[...]
    mu = dis[:, None] * jnp.dot(s2, wmu_ref[...],
                                preferred_element_type=jnp.float32)
    lg = jnp.minimum(dis[:, None] * jnp.dot(s2, wls_ref[...],
                                            preferred_element_type=jnp.float32),
                     MAXLS)
    z_ref[...] = mu + eps_ref[...] * jnp.exp(lg)


def kernel(x, edge_index, eps, W1, W_mu, W_logstd):
    row = edge_index[0]
    col = edge_index[1]

    degf = _deg_kernel(col)
    deg2 = degf.reshape(NC, N, 16)[:, :, 0]

    t, dis = pl.pallas_call(
        _tc1_body,
        out_shape=(jax.ShapeDtypeStruct((N, H), jnp.float32),
                   jax.ShapeDtypeStruct((N,), jnp.float32)),
    )(deg2, x, W1)

    zerosF = jnp.zeros((N, H), jnp.float32)
    s1p = _prop128(t, row, col, zerosF)

    g = pl.pallas_call(
        _tc2_body,
        out_shape=jax.ShapeDtypeStruct((N, H), jnp.float32),
    )(s1p, dis)

    s2p = _prop128(g, row, col, zerosF)

    z = pl.pallas_call(
        _tc3_body,
        out_shape=jax.ShapeDtypeStruct((N, L), jnp.float32),
    )(s2p, dis, eps, W_mu, W_logstd)

    return _decode_kernel(z, row, col)
